# Initial kernel scaffold; baseline (speedup 1.0000x reference)
import jax
import jax.numpy as jnp
from jax.experimental import pallas as pl


def kernel(x, table, W, b):
    raise NotImplementedError("write your pallas kernel here")


# speedup vs baseline: 2.5545x; 2.5545x over previous
import dataclasses
import functools

import jax
import jax.numpy as jnp
from jax import lax
from jax.experimental import pallas as pl
from jax.experimental.pallas import tpu as pltpu
from jax.experimental.pallas import tpu_sc as plsc

N_EMB = 1000000
EMB_DIM = 64
CLS = 2
CLS_PAD = 16
BATCH = 16384
HIST = 200

NC = 2
NS = 16
L = 16
NW = NC * NS
B_PER_W = BATCH // NW
G = 16
GROUPS = B_PER_W // G
IDX_PER_G = G * HIST
IDX_W = 128
IDX_ROWS = IDX_PER_G // IDX_W
GPSB = 8
SB = GROUPS // GPSB
SB_ROWS = GPSB * IDX_ROWS
UNROLL = 8

MM_BLK = 8000


def _mm_body(t_ref, w_ref, b_ref, o_ref):
    acc = lax.dot_general(
        t_ref[...], w_ref[...],
        dimension_numbers=(((1,), (1,)), ((), ())),
        preferred_element_type=jnp.float32,
    )
    o_ref[...] = (acc + b_ref[...]) * (1.0 / HIST)


def _fold_table(table, Wp, bp):
    return pl.pallas_call(
        _mm_body,
        grid=(N_EMB // MM_BLK,),
        in_specs=[
            pl.BlockSpec((MM_BLK, EMB_DIM), lambda i: (i, 0)),
            pl.BlockSpec((CLS_PAD, EMB_DIM), lambda i: (0, 0)),
            pl.BlockSpec((1, CLS_PAD), lambda i: (0, 0)),
        ],
        out_specs=pl.BlockSpec((MM_BLK, CLS_PAD), lambda i: (i, 0)),
        out_shape=jax.ShapeDtypeStruct((N_EMB, CLS_PAD), jnp.float32),
    )(table, Wp, bp.reshape(1, CLS_PAD))


def _sc_gather_sum(x_rows, tableW):
    mesh = plsc.VectorSubcoreMesh(core_axis_name="c", subcore_axis_name="s")
    cp = pltpu.CompilerParams(
        needs_layout_passes=False,
        use_tc_tiling_on_sc=False,
    )

    @functools.partial(
        pl.kernel,
        out_type=jax.ShapeDtypeStruct((BATCH, CLS), jnp.float32),
        mesh=mesh,
        compiler_params=cp,
        scratch_types=[
            pltpu.VMEM((SB_ROWS, IDX_W), jnp.int32),
            pltpu.VMEM((IDX_PER_G, CLS_PAD), jnp.float32),
            pltpu.VMEM((B_PER_W, CLS), jnp.float32),
            pltpu.SemaphoreType.DMA,
        ],
    )
    def k(x_hbm, tw_hbm, out_hbm, idx_v, rows_v, out_v, sem):
        wid = lax.axis_index("s") * NC + lax.axis_index("c")
        idx_row_base = wid * (B_PER_W * HIST // IDX_W)
        lanes = lax.iota(jnp.int32, L)
        col0 = jnp.zeros((L,), jnp.int32)
        col1 = jnp.ones((L,), jnp.int32)

        @pl.loop(0, SB)
        def _(sb):
            pltpu.sync_copy(
                x_hbm.at[pl.ds(idx_row_base + sb * SB_ROWS, SB_ROWS)], idx_v)

            @pl.loop(0, GPSB)
            def _(gg):
                copies = [
                    pltpu.async_copy(
                        tw_hbm.at[idx_v.at[gg * IDX_ROWS + j]],
                        rows_v.at[pl.ds(j * IDX_W, IDX_W)],
                        sem,
                    )
                    for j in range(IDX_ROWS)
                ]
                for cpy in copies:
                    cpy.wait()

                def body(i, accs):
                    a0, a1 = accs
                    for kk in range(UNROLL):
                        r = lanes * HIST + (i * UNROLL + kk)
                        a0 = a0 + plsc.load_gather(rows_v, [r, col0])
                        a1 = a1 + plsc.load_gather(rows_v, [r, col1])
                    return (a0, a1)

                z = jnp.zeros((L,), jnp.float32)
                a0, a1 = lax.fori_loop(0, HIST // UNROLL, body, (z, z))
                row_idx = sb * (GPSB * G) + gg * G + lanes
                plsc.store_scatter(out_v, [row_idx, col0], a0)
                plsc.store_scatter(out_v, [row_idx, col1], a1)

        pltpu.sync_copy(out_v, out_hbm.at[pl.ds(wid * B_PER_W, B_PER_W)])

    return k(x_rows, tableW)


def kernel(x, table, W, b):
    x_rows = x.reshape(BATCH * HIST // IDX_W, IDX_W).astype(jnp.int32)
    Wp = jnp.zeros((CLS_PAD, EMB_DIM), jnp.float32).at[:CLS].set(W)
    bp = jnp.zeros((CLS_PAD,), jnp.float32).at[:CLS].set(b)
    tableW = _fold_table(table, Wp, bp)
    return _sc_gather_sum(x_rows, tableW)


# speedup vs baseline: 3.7016x; 1.4491x over previous
import dataclasses
import functools

import jax
import jax.numpy as jnp
from jax import lax
from jax.experimental import pallas as pl
from jax.experimental.pallas import tpu as pltpu
from jax.experimental.pallas import tpu_sc as plsc

N_EMB = 1000000
EMB_DIM = 64
CLS = 2
CLS_PAD = 16
BATCH = 16384
HIST = 200

NC = 2
NS = 16
L = 16
NW = NC * NS
B_PER_W = BATCH // NW
G = 16
GROUPS = B_PER_W // G
IDX_PER_G = G * HIST
IDX_W = 128
IDX_ROWS = IDX_PER_G // IDX_W
GPSB = 8
SB = GROUPS // GPSB
SB_ROWS = GPSB * IDX_ROWS
UNROLL = 8

MM_BLK = 8192


def _mm_body(t_ref, w_ref, b_ref, o_ref):
    acc = lax.dot_general(
        t_ref[...], w_ref[...],
        dimension_numbers=(((0,), (1,)), ((), ())),
        preferred_element_type=jnp.float32,
    )
    o_ref[...] = (acc + b_ref[...]) * (1.0 / HIST)


def _fold_table(tableT, Wp, bp):
    return pl.pallas_call(
        _mm_body,
        grid=(pl.cdiv(N_EMB, MM_BLK),),
        in_specs=[
            pl.BlockSpec((EMB_DIM, MM_BLK), lambda i: (0, i)),
            pl.BlockSpec((CLS_PAD, EMB_DIM), lambda i: (0, 0)),
            pl.BlockSpec((1, CLS_PAD), lambda i: (0, 0)),
        ],
        out_specs=pl.BlockSpec((MM_BLK, CLS_PAD), lambda i: (i, 0)),
        out_shape=jax.ShapeDtypeStruct((N_EMB, CLS_PAD), jnp.float32),
    )(tableT, Wp, bp.reshape(1, CLS_PAD))


def _sc_gather_sum(x_rows, tableW):
    mesh = plsc.VectorSubcoreMesh(core_axis_name="c", subcore_axis_name="s")
    cp = pltpu.CompilerParams(
        needs_layout_passes=False,
        use_tc_tiling_on_sc=False,
    )

    @functools.partial(
        pl.kernel,
        out_type=jax.ShapeDtypeStruct((BATCH, CLS), jnp.float32),
        mesh=mesh,
        compiler_params=cp,
        scratch_types=[
            pltpu.VMEM((SB_ROWS, IDX_W), jnp.int32),
            pltpu.VMEM((IDX_PER_G, CLS_PAD), jnp.float32),
            pltpu.VMEM((B_PER_W, CLS), jnp.float32),
            pltpu.SemaphoreType.DMA,
        ],
    )
    def k(x_hbm, tw_hbm, out_hbm, idx_v, rows_v, out_v, sem):
        wid = lax.axis_index("s") * NC + lax.axis_index("c")
        idx_row_base = wid * (B_PER_W * HIST // IDX_W)
        lanes = lax.iota(jnp.int32, L)
        col0 = jnp.zeros((L,), jnp.int32)
        col1 = jnp.ones((L,), jnp.int32)

        @pl.loop(0, SB)
        def _(sb):
            pltpu.sync_copy(
                x_hbm.at[pl.ds(idx_row_base + sb * SB_ROWS, SB_ROWS)], idx_v)

            @pl.loop(0, GPSB)
            def _(gg):
                copies = [
                    pltpu.async_copy(
                        tw_hbm.at[idx_v.at[gg * IDX_ROWS + j]],
                        rows_v.at[pl.ds(j * IDX_W, IDX_W)],
                        sem,
                    )
                    for j in range(IDX_ROWS)
                ]
                for cpy in copies:
                    cpy.wait()

                def body(i, accs):
                    a0, a1 = accs
                    for kk in range(UNROLL):
                        r = lanes * HIST + (i * UNROLL + kk)
                        a0 = a0 + plsc.load_gather(rows_v, [r, col0])
                        a1 = a1 + plsc.load_gather(rows_v, [r, col1])
                    return (a0, a1)

                z = jnp.zeros((L,), jnp.float32)
                a0, a1 = lax.fori_loop(0, HIST // UNROLL, body, (z, z))
                row_idx = sb * (GPSB * G) + gg * G + lanes
                plsc.store_scatter(out_v, [row_idx, col0], a0)
                plsc.store_scatter(out_v, [row_idx, col1], a1)

        pltpu.sync_copy(out_v, out_hbm.at[pl.ds(wid * B_PER_W, B_PER_W)])

    return k(x_rows, tableW)


def kernel(x, table, W, b):
    x_rows = x.reshape(BATCH * HIST // IDX_W, IDX_W).astype(jnp.int32)
    Wp = jnp.zeros((CLS_PAD, EMB_DIM), jnp.float32).at[:CLS].set(W)
    bp = jnp.zeros((CLS_PAD,), jnp.float32).at[:CLS].set(b)
    tableW = _fold_table(table.T, Wp, bp)
    return _sc_gather_sum(x_rows, tableW)


# speedup vs baseline: 7.3103x; 1.9749x over previous
import dataclasses
import functools

import jax
import jax.numpy as jnp
from jax import lax
from jax.experimental import pallas as pl
from jax.experimental.pallas import tpu as pltpu
from jax.experimental.pallas import tpu_sc as plsc

N_EMB = 1000000
EMB_DIM = 64
CLS = 2
CLS_PAD = 16
BATCH = 16384
HIST = 200

NC = 2
NS = 16
L = 16
NW = NC * NS
B_PER_W = BATCH // NW
G = 16
GROUPS = B_PER_W // G
IDX_PER_G = G * HIST
IDX_W = 128
IDX_ROWS = IDX_PER_G // IDX_W
GPSB = 8
SB = GROUPS // GPSB
SB_ROWS = GPSB * IDX_ROWS
UNROLL = 8

MM_BLK = 8192
MM_SUB = MM_BLK // 8
MM_GRID = -(-N_EMB // MM_BLK)
TW_ROWS = MM_GRID * MM_BLK


def _mm_body(t_ref, w_ref, b_ref, o_ref):
    for j in range(8):
        acc = lax.dot_general(
            t_ref[:, j * MM_SUB:(j + 1) * MM_SUB], w_ref[...],
            dimension_numbers=(((0,), (1,)), ((), ())),
            preferred_element_type=jnp.float32,
        )
        o_ref[:, j * CLS_PAD:(j + 1) * CLS_PAD] = (
            (acc + b_ref[...]) * (1.0 / HIST))


def _fold_table(tableT, Wp, bp):
    return pl.pallas_call(
        _mm_body,
        grid=(MM_GRID,),
        in_specs=[
            pl.BlockSpec((EMB_DIM, MM_BLK), lambda i: (0, i)),
            pl.BlockSpec((CLS_PAD, EMB_DIM), lambda i: (0, 0)),
            pl.BlockSpec((1, CLS_PAD), lambda i: (0, 0)),
        ],
        out_specs=pl.BlockSpec((MM_SUB, 128), lambda i: (i, 0)),
        out_shape=jax.ShapeDtypeStruct((MM_GRID * MM_SUB, 128), jnp.float32),
    )(tableT, Wp, bp.reshape(1, CLS_PAD))


def _sc_gather_sum(x_rows, tableW):
    mesh = plsc.VectorSubcoreMesh(core_axis_name="c", subcore_axis_name="s")
    cp = pltpu.CompilerParams(
        needs_layout_passes=False,
        use_tc_tiling_on_sc=False,
    )

    @functools.partial(
        pl.kernel,
        out_type=jax.ShapeDtypeStruct((BATCH, CLS), jnp.float32),
        mesh=mesh,
        compiler_params=cp,
        scratch_types=[
            pltpu.VMEM((SB_ROWS, IDX_W), jnp.int32),
            pltpu.VMEM((IDX_PER_G, CLS_PAD), jnp.float32),
            pltpu.VMEM((B_PER_W, CLS), jnp.float32),
            pltpu.SemaphoreType.DMA,
        ],
    )
    def k(x_hbm, tw_hbm, out_hbm, idx_v, rows_v, out_v, sem):
        wid = lax.axis_index("s") * NC + lax.axis_index("c")
        idx_row_base = wid * (B_PER_W * HIST // IDX_W)
        lanes = lax.iota(jnp.int32, L)
        col0 = jnp.zeros((L,), jnp.int32)
        col1 = jnp.ones((L,), jnp.int32)

        @pl.loop(0, SB)
        def _(sb):
            pltpu.sync_copy(
                x_hbm.at[pl.ds(idx_row_base + sb * SB_ROWS, SB_ROWS)], idx_v)

            @pl.loop(0, SB_ROWS)
            def _(row):
                for kk in range(IDX_W // L):
                    v = idx_v[row, pl.ds(kk * L, L)]
                    idx_v[row, pl.ds(kk * L, L)] = (
                        (v & -8192) | ((v & 1023) << 3) | ((v >> 10) & 7))

            @pl.loop(0, GPSB)
            def _(gg):
                copies = [
                    pltpu.async_copy(
                        tw_hbm.at[idx_v.at[gg * IDX_ROWS + j]],
                        rows_v.at[pl.ds(j * IDX_W, IDX_W)],
                        sem,
                    )
                    for j in range(IDX_ROWS)
                ]
                for cpy in copies:
                    cpy.wait()

                def body(i, accs):
                    a0, a1 = accs
                    for kk in range(UNROLL):
                        r = lanes * HIST + (i * UNROLL + kk)
                        a0 = a0 + plsc.load_gather(rows_v, [r, col0])
                        a1 = a1 + plsc.load_gather(rows_v, [r, col1])
                    return (a0, a1)

                z = jnp.zeros((L,), jnp.float32)
                a0, a1 = lax.fori_loop(0, HIST // UNROLL, body, (z, z))
                row_idx = sb * (GPSB * G) + gg * G + lanes
                plsc.store_scatter(out_v, [row_idx, col0], a0)
                plsc.store_scatter(out_v, [row_idx, col1], a1)

        pltpu.sync_copy(out_v, out_hbm.at[pl.ds(wid * B_PER_W, B_PER_W)])

    return k(x_rows, tableW)


def kernel(x, table, W, b):
    x_rows = x.reshape(BATCH * HIST // IDX_W, IDX_W).astype(jnp.int32)
    Wp = jnp.zeros((CLS_PAD, EMB_DIM), jnp.float32).at[:CLS].set(W)
    bp = jnp.zeros((CLS_PAD,), jnp.float32).at[:CLS].set(b)
    tableW = _fold_table(table.T, Wp, bp).reshape(TW_ROWS, CLS_PAD)
    return _sc_gather_sum(x_rows, tableW)
